# Initial kernel scaffold; baseline (speedup 1.0000x reference)
import jax
import jax.numpy as jnp
from jax.experimental import pallas as pl


def kernel(features, coords):
    raise NotImplementedError("write your pallas kernel here")


# speedup vs baseline: 499.8486x; 499.8486x over previous
import functools
import jax
import jax.numpy as jnp
from jax import lax
from jax.experimental import pallas as pl
from jax.experimental.pallas import tpu as pltpu
from jax.experimental.pallas import tpu_sc as plsc

_LAM = 0.5
_SIG2 = 100.0 ** 2
_KC = 8
_NEG = float("-inf")


def _tc_body(init_ref, feat_ref, coord_ref, sim_ref, ids_ref, conn_ref,
             cent_ref, conn2_ref, fn_ref):
    feat = feat_ref[0]
    n = feat.shape[0]
    d = feat.shape[1]

    nrm2 = jnp.sum(feat * feat, axis=1, keepdims=True)
    nrm = jnp.maximum(jnp.sqrt(nrm2), 1e-12)
    fnorm = feat / nrm
    fn_ref[...] = fnorm
    c0 = coord_ref[0, :, 0]
    c1 = coord_ref[0, :, 1]

    def simr(r, carry):
        base = pl.multiple_of(r * 128, 128)
        fr = fn_ref[pl.ds(base, 128), :]
        sem = lax.dot_general(fr, fnorm, (((1,), (1,)), ((), ())),
                              preferred_element_type=jnp.float32)
        cr = coord_ref[0, pl.ds(base, 128), :]
        c0r = cr[:, 0]
        c1r = cr[:, 1]
        dx = c0r[:, None] - c0[None, :]
        dy = c1r[:, None] - c1[None, :]
        t = dx * dx + dy * dy
        s = jnp.sqrt(t)
        sp = jnp.exp(-(s * s) / _SIG2)
        sim_ref[0, pl.ds(base, 128), :] = sem + _LAM * sp
        return carry
    lax.fori_loop(0, n // 128, simr, 0)

    for k in range(_KC):
        cent_ref[pl.ds(k, 1), :] = feat_ref[0, pl.ds(init_ref[0, 0, k], 1), :]

    def kiter(it, ids_c):
        best = jnp.full((n,), jnp.inf, jnp.float32)
        bid = jnp.zeros((n,), jnp.int32)
        for c in range(_KC):
            diff = feat - cent_ref[pl.ds(c, 1), :]
            d2 = jnp.sum(diff * diff, axis=1)
            dd = jnp.sqrt(d2)
            upd = dd < best
            best = jnp.where(upd, dd, best)
            bid = jnp.where(upd, c, bid)
        for c in range(_KC):
            m = bid == c
            cnt = jnp.sum(m.astype(jnp.int32))
            ssum = jnp.sum(jnp.where(m[:, None], feat, 0.0), axis=0)
            mean = ssum / jnp.maximum(cnt, 1).astype(jnp.float32)
            old = cent_ref[pl.ds(c, 1), :]
            cent_ref[pl.ds(c, 1), :] = jnp.where(cnt > 0, mean[None, :], old)
        return bid
    ids = lax.fori_loop(0, 10, kiter, jnp.zeros((n,), jnp.int32))
    ids_ref[0, 0, :] = ids

    onehot = (ids[:, None] ==
              lax.broadcasted_iota(jnp.int32, (1, _KC), 1)).astype(jnp.float32)

    def connr(r, carry):
        base = pl.multiple_of(r * 128, 128)
        chunk = sim_ref[0, pl.ds(base, 128), :]
        slab = lax.dot_general(chunk, onehot, (((1,), (0,)), ((), ())),
                               preferred_element_type=jnp.float32)
        conn2_ref[pl.ds(base, 128), :] = slab
        return carry
    lax.fori_loop(0, n // 128, connr, 0)

    connt = jnp.transpose(conn2_ref[...])
    for c in range(_KC):
        m = ids == c
        conn_ref[0, c, :] = jnp.where(m, connt[c], _NEG)


def _init_indices(batch, n):
    rows = []
    for b in range(batch):
        key = jax.random.fold_in(jax.random.key(123), b)
        rows.append(jax.random.permutation(key, n)[:_KC])
    return jnp.stack(rows).astype(jnp.int32).reshape(batch, 1, _KC)


def _tc_stage(features, coords):
    B, N, D = features.shape
    init_idx = _init_indices(B, N)
    sim, ids, conn = pl.pallas_call(
        _tc_body,
        grid=(B,),
        in_specs=[
            pl.BlockSpec((1, 1, _KC), lambda b: (b, 0, 0),
                         memory_space=pltpu.SMEM),
            pl.BlockSpec((1, N, D), lambda b: (b, 0, 0)),
            pl.BlockSpec((1, N, 2), lambda b: (b, 0, 0)),
        ],
        out_specs=[
            pl.BlockSpec((1, N, N), lambda b: (b, 0, 0)),
            pl.BlockSpec((1, 1, N), lambda b: (b, 0, 0)),
            pl.BlockSpec((1, _KC, N), lambda b: (b, 0, 0)),
        ],
        out_shape=[
            jax.ShapeDtypeStruct((B, N, N), jnp.float32),
            jax.ShapeDtypeStruct((B, 1, N), jnp.int32),
            jax.ShapeDtypeStruct((B, _KC, N), jnp.float32),
        ],
        scratch_shapes=[
            pltpu.VMEM((_KC, D), jnp.float32),
            pltpu.VMEM((N, _KC), jnp.float32),
            pltpu.VMEM((N, D), jnp.float32),
        ],
    )(init_idx, features, coords)
    return sim, ids, conn


def _sc_greedy(feat4, sim4, conn32, ids, B, N, D):
    mesh = plsc.VectorSubcoreMesh(core_axis_name="c", subcore_axis_name="s",
                                  num_cores=2, num_subcores=16)
    nch = N // 16

    @functools.partial(
        pl.kernel,
        mesh=mesh,
        compiler_params=pltpu.CompilerParams(needs_layout_passes=False),
        out_type=[
            jax.ShapeDtypeStruct((B, N, 128), jnp.float32),
            jax.ShapeDtypeStruct((B, N), jnp.int32),
        ],
        scratch_types=[
            pltpu.VMEM((N,), jnp.int32),
            pltpu.VMEM((N + 16,), jnp.int32),
            pltpu.VMEM((N,), jnp.float32),
            pltpu.VMEM((N,), jnp.float32),
            pltpu.VMEM((N,), jnp.float32),
            pltpu.VMEM((N,), jnp.int32),
            pltpu.VMEM((16,), jnp.int32),
            pltpu.VMEM((_KC * N,), jnp.int32),
            pltpu.VMEM((16 * 16,), jnp.int32),
            pltpu.VMEM((N,), jnp.int32),
            pltpu.VMEM((N,), jnp.int32),
            pltpu.VMEM((128, 128), jnp.float32),
            pltpu.VMEM_SHARED((16 * N,), jnp.int32),
            pltpu.VMEM_SHARED((16 * 16,), jnp.int32),
            pltpu.SemaphoreType.DMA,
        ],
    )
    def sc_kernel(feat4_hbm, sim4_hbm, conn_hbm, ids_hbm, reord_hbm,
                  order_hbm, ids_v, mem_v, conn_v, row_v, alive_v, ordseg_v,
                  offs_v, segs_v, osz_v, order_v, bgidx_v, featbuf_v,
                  seg_sh, osz_sh, sem):
        core = lax.axis_index("c")
        sub = lax.axis_index("s")
        b = core * 2 + sub // _KC
        c = sub % _KC
        iota16 = lax.broadcasted_iota(jnp.int32, (16,), 0)
        zeros_i = jnp.zeros((16,), jnp.int32)
        neg16 = jnp.full((16,), _NEG, jnp.float32)
        lane0 = iota16 == 0
        shufs = [iota16 ^ 8, iota16 ^ 4, iota16 ^ 2, iota16 ^ 1]

        def _shuf(x, idx):
            return x.at[idx].get(mode="promise_in_bounds")

        def _pcnt(m):
            r = plsc.all_reduce_population_count(m)
            return r if getattr(r, "ndim", 0) == 0 else r[0]

        def _argmax2(v, i):
            for s in shufs:
                ov, oi = _shuf(v, s), _shuf(i, s)
                better = (ov > v) | ((ov == v) & (oi < i))
                v = jnp.where(better, ov, v)
                i = jnp.where(better, oi, i)
            return v, i

        def _argmax3(v, i, o):
            for s in shufs:
                ov, oi, oo = _shuf(v, s), _shuf(i, s), _shuf(o, s)
                better = (ov > v) | ((ov == v) & (oi < i))
                v = jnp.where(better, ov, v)
                i = jnp.where(better, oi, i)
                o = jnp.where(better, oo, o)
            return v, i, o

        pltpu.sync_copy(ids_hbm.at[b], ids_v)
        pltpu.sync_copy(conn_hbm.at[b * _KC + c], conn_v)

        def zinit(k, carry):
            mem_v[pl.ds(k * 16, 16)] = zeros_i
            return carry
        lax.fori_loop(0, nch + 1, zinit, 0)

        def comp(k, st):
            size, offset = st
            v = ids_v[pl.ds(k * 16, 16)]
            meq = v == c
            idxvec = k * 16 + iota16
            ones = jnp.where(meq, 1, 0)
            pre = ones
            for s in (1, 2, 4, 8):
                pre = pre + jnp.where(iota16 >= s,
                                      _shuf(pre, (iota16 - s) & 15), 0)
            pos = size + pre - ones
            plsc.store_scatter(mem_v, [pos], idxvec, mask=meq)
            size = size + _pcnt(meq)
            offset = offset + _pcnt(v < c)
            return size, offset
        size, offset = lax.fori_loop(0, nch, comp,
                                     (jnp.int32(0), jnp.int32(0)))

        def cch(k, st):
            bv, bi = st
            v = conn_v[pl.ds(k * 16, 16)]
            pos = k * 16 + iota16
            m = v > bv
            return jnp.where(m, v, bv), jnp.where(m, pos, bi)
        bv, bi = lax.fori_loop(0, nch, cch, (neg16, zeros_i))
        _, sti = _argmax2(bv, bi)
        start = sti[0]

        def ainit(k, carry):
            cpos = k * 16 + iota16
            mvals = mem_v[pl.ds(k * 16, 16)]
            a = jnp.where((cpos < size) & (mvals != start), 0.0, _NEG)
            alive_v[pl.ds(k * 16, 16)] = a.astype(jnp.float32)
            return carry
        lax.fori_loop(0, nch, ainit, 0)
        plsc.store_scatter(ordseg_v, [zeros_i], zeros_i + start, mask=lane0)

        def step(i, cur):
            pltpu.sync_copy(sim4_hbm.at[b * N + cur], row_v)
            nck = (size + 15) // 16

            def ch(j, st):
                sbv, sbi, sbo = st
                idxs = mem_v[pl.ds(j * 16, 16)]
                av = alive_v[pl.ds(j * 16, 16)]
                vals = plsc.load_gather(row_v, [idxs]) + av
                cpos = j * 16 + iota16
                m = vals > sbv
                return (jnp.where(m, vals, sbv), jnp.where(m, cpos, sbi),
                        jnp.where(m, idxs, sbo))
            sbv, sbi, sbo = lax.fori_loop(0, nck, ch,
                                          (neg16, zeros_i, zeros_i))
            _, pv, nv = _argmax3(sbv, sbi, sbo)
            p = pv[0]
            nxt = nv[0]
            plsc.store_scatter(alive_v, [zeros_i + p], neg16, mask=lane0)
            plsc.store_scatter(ordseg_v, [zeros_i + i], zeros_i + nxt,
                               mask=lane0)
            return nxt
        lax.fori_loop(1, size, step, start)

        w = jnp.where(iota16 == 0, offset, jnp.where(iota16 == 1, size, 0))
        offs_v[...] = w.astype(jnp.int32)
        pltpu.sync_copy(offs_v, osz_sh.at[pl.ds(sub * 16, 16)])
        pltpu.sync_copy(ordseg_v, seg_sh.at[pl.ds(sub * N, N)])
        plsc.subcore_barrier()

        @pl.when(c == 0)
        def _():
            pltpu.sync_copy(osz_sh, osz_v)
            pltpu.sync_copy(seg_sh.at[pl.ds(sub * N, _KC * N)], segs_v)
            offs = []
            for cc in range(_KC):
                vv = osz_v[pl.ds((sub + cc) * 16, 16)]
                offs.append(vv[0])
            ovec = zeros_i
            for cc in range(_KC):
                ovec = jnp.where(iota16 == cc, offs[cc], ovec)
            offs_v[...] = ovec

            def asm(k, carry):
                jvec = k * 16 + iota16
                cnt = zeros_i
                for cc in range(_KC):
                    cnt = cnt + jnp.where(jvec >= offs[cc], 1, 0)
                cj = cnt - 1
                offj = plsc.load_gather(offs_v, [cj])
                src = cj * N + jvec - offj
                val = plsc.load_gather(segs_v, [src])
                order_v[pl.ds(k * 16, 16)] = val
                bgidx_v[pl.ds(k * 16, 16)] = val + b * N
                return carry
            lax.fori_loop(0, nch, asm, 0)

            for r in range(N // 128):
                pltpu.async_copy(
                    feat4_hbm.at[bgidx_v.at[pl.ds(r * 128, 128)]],
                    featbuf_v, sem).wait()
                pltpu.sync_copy(featbuf_v,
                                reord_hbm.at[b, pl.ds(r * 128, 128)])
            pltpu.sync_copy(order_v, order_hbm.at[b])

    return sc_kernel(feat4, sim4, conn32, ids)


def kernel(features, coords):
    B, N, D = features.shape
    sim, ids3, conn = _tc_stage(features, coords)
    feat4p = jnp.pad(features.reshape(B * N, D), ((0, 0), (0, 128 - D)))
    reord, order = _sc_greedy(
        feat4p, sim.reshape(B * N, N),
        conn.reshape(B * _KC, N), ids3.reshape(B, N), B, N, D)
    return reord[:, :, :D], order


# speedup vs baseline: 565.8823x; 1.1321x over previous
import functools
import jax
import jax.numpy as jnp
from jax import lax
from jax.experimental import pallas as pl
from jax.experimental.pallas import tpu as pltpu
from jax.experimental.pallas import tpu_sc as plsc

_LAM = 0.5
_SIG2 = 100.0 ** 2
_KC = 8
_NEG = float("-inf")
_CAP = 288


def _tc_body(init_ref, feat_ref, coord_ref, sim_ref, ids_ref, conn_ref,
             cent_ref, conn2_ref, fn_ref):
    feat = feat_ref[0]
    n = feat.shape[0]
    d = feat.shape[1]

    nrm2 = jnp.sum(feat * feat, axis=1, keepdims=True)
    nrm = jnp.maximum(jnp.sqrt(nrm2), 1e-12)
    fnorm = feat / nrm
    fn_ref[...] = fnorm
    c0 = coord_ref[0, :, 0]
    c1 = coord_ref[0, :, 1]

    def simr(r, carry):
        base = pl.multiple_of(r * 128, 128)
        fr = fn_ref[pl.ds(base, 128), :]
        sem = lax.dot_general(fr, fnorm, (((1,), (1,)), ((), ())),
                              preferred_element_type=jnp.float32)
        cr = coord_ref[0, pl.ds(base, 128), :]
        c0r = cr[:, 0]
        c1r = cr[:, 1]
        dx = c0r[:, None] - c0[None, :]
        dy = c1r[:, None] - c1[None, :]
        t = dx * dx + dy * dy
        s = jnp.sqrt(t)
        sp = jnp.exp(-(s * s) / _SIG2)
        sim_ref[0, pl.ds(base, 128), :] = sem + _LAM * sp
        return carry
    lax.fori_loop(0, n // 128, simr, 0)

    for k in range(_KC):
        cent_ref[pl.ds(k, 1), :] = feat_ref[0, pl.ds(init_ref[0, 0, k], 1), :]

    def kiter(it, ids_c):
        best = jnp.full((n,), jnp.inf, jnp.float32)
        bid = jnp.zeros((n,), jnp.int32)
        for c in range(_KC):
            diff = feat - cent_ref[pl.ds(c, 1), :]
            d2 = jnp.sum(diff * diff, axis=1)
            dd = jnp.sqrt(d2)
            upd = dd < best
            best = jnp.where(upd, dd, best)
            bid = jnp.where(upd, c, bid)
        for c in range(_KC):
            m = bid == c
            cnt = jnp.sum(m.astype(jnp.int32))
            ssum = jnp.sum(jnp.where(m[:, None], feat, 0.0), axis=0)
            mean = ssum / jnp.maximum(cnt, 1).astype(jnp.float32)
            old = cent_ref[pl.ds(c, 1), :]
            cent_ref[pl.ds(c, 1), :] = jnp.where(cnt > 0, mean[None, :], old)
        return bid
    ids = lax.fori_loop(0, 10, kiter, jnp.zeros((n,), jnp.int32))
    ids_ref[0, 0, :] = ids

    onehot = (ids[:, None] ==
              lax.broadcasted_iota(jnp.int32, (1, _KC), 1)).astype(jnp.float32)

    def connr(r, carry):
        base = pl.multiple_of(r * 128, 128)
        chunk = sim_ref[0, pl.ds(base, 128), :]
        slab = lax.dot_general(chunk, onehot, (((1,), (0,)), ((), ())),
                               preferred_element_type=jnp.float32)
        conn2_ref[pl.ds(base, 128), :] = slab
        return carry
    lax.fori_loop(0, n // 128, connr, 0)

    connt = jnp.transpose(conn2_ref[...])
    for c in range(_KC):
        m = ids == c
        conn_ref[0, c, :] = jnp.where(m, connt[c], _NEG)


def _init_indices(batch, n):
    rows = []
    for b in range(batch):
        key = jax.random.fold_in(jax.random.key(123), b)
        rows.append(jax.random.permutation(key, n)[:_KC])
    return jnp.stack(rows).astype(jnp.int32).reshape(batch, 1, _KC)


def _tc_stage(features, coords):
    B, N, D = features.shape
    init_idx = _init_indices(B, N)
    sim, ids, conn = pl.pallas_call(
        _tc_body,
        grid=(B,),
        in_specs=[
            pl.BlockSpec((1, 1, _KC), lambda b: (b, 0, 0),
                         memory_space=pltpu.SMEM),
            pl.BlockSpec((1, N, D), lambda b: (b, 0, 0)),
            pl.BlockSpec((1, N, 2), lambda b: (b, 0, 0)),
        ],
        out_specs=[
            pl.BlockSpec((1, N, N), lambda b: (b, 0, 0)),
            pl.BlockSpec((1, 1, N), lambda b: (b, 0, 0)),
            pl.BlockSpec((1, _KC, N), lambda b: (b, 0, 0)),
        ],
        out_shape=[
            jax.ShapeDtypeStruct((B, N, N), jnp.float32),
            jax.ShapeDtypeStruct((B, 1, N), jnp.int32),
            jax.ShapeDtypeStruct((B, _KC, N), jnp.float32),
        ],
        scratch_shapes=[
            pltpu.VMEM((_KC, D), jnp.float32),
            pltpu.VMEM((N, _KC), jnp.float32),
            pltpu.VMEM((N, D), jnp.float32),
        ],
    )(init_idx, features, coords)
    return sim, ids, conn


def _sc_greedy(feat4, sim4, conn32, ids, B, N, D):
    mesh = plsc.VectorSubcoreMesh(core_axis_name="c", subcore_axis_name="s",
                                  num_cores=2, num_subcores=16)
    nch = N // 16

    @functools.partial(
        pl.kernel,
        mesh=mesh,
        compiler_params=pltpu.CompilerParams(needs_layout_passes=False),
        out_type=[
            jax.ShapeDtypeStruct((B, N, 128), jnp.float32),
            jax.ShapeDtypeStruct((B, N), jnp.int32),
        ],
        scratch_types=[
            pltpu.VMEM((N,), jnp.int32),
            pltpu.VMEM((N + 16,), jnp.int32),
            pltpu.VMEM((N,), jnp.float32),
            pltpu.VMEM((N,), jnp.float32),
            pltpu.VMEM((N,), jnp.float32),
            pltpu.VMEM((N,), jnp.int32),
            pltpu.VMEM((16,), jnp.int32),
            pltpu.VMEM((_KC * N,), jnp.int32),
            pltpu.VMEM((16 * 16,), jnp.int32),
            pltpu.VMEM((N,), jnp.int32),
            pltpu.VMEM((N,), jnp.int32),
            pltpu.VMEM((128, 128), jnp.float32),
            pltpu.VMEM((_CAP * _CAP,), jnp.float32),
            pltpu.VMEM((N,), jnp.float32),
            pltpu.VMEM_SHARED((16 * N,), jnp.int32),
            pltpu.VMEM_SHARED((16 * 16,), jnp.int32),
            pltpu.SemaphoreType.DMA,
            pltpu.SemaphoreType.DMA,
            pltpu.SemaphoreType.DMA,
        ],
    )
    def sc_kernel(feat4_hbm, sim4_hbm, conn_hbm, ids_hbm, reord_hbm,
                  order_hbm, ids_v, mem_v, conn_v, row_v, alive_v, ordseg_v,
                  offs_v, segs_v, osz_v, order_v, bgidx_v, featbuf_v,
                  subm_v, rowb_v, seg_sh, osz_sh, sem, semA, semB):
        core = lax.axis_index("c")
        sub = lax.axis_index("s")
        b = core * 2 + sub // _KC
        c = sub % _KC
        iota16 = lax.broadcasted_iota(jnp.int32, (16,), 0)
        zeros_i = jnp.zeros((16,), jnp.int32)
        neg16 = jnp.full((16,), _NEG, jnp.float32)
        lane0 = iota16 == 0
        shufs = [iota16 ^ 8, iota16 ^ 4, iota16 ^ 2, iota16 ^ 1]

        def _shuf(x, idx):
            return x.at[idx].get(mode="promise_in_bounds")

        def _pcnt(m):
            r = plsc.all_reduce_population_count(m)
            return r if getattr(r, "ndim", 0) == 0 else r[0]

        def _argmax2(v, i):
            for s in shufs:
                ov, oi = _shuf(v, s), _shuf(i, s)
                better = (ov > v) | ((ov == v) & (oi < i))
                v = jnp.where(better, ov, v)
                i = jnp.where(better, oi, i)
            return v, i

        def _argmax3(v, i, o):
            for s in shufs:
                ov, oi, oo = _shuf(v, s), _shuf(i, s), _shuf(o, s)
                better = (ov > v) | ((ov == v) & (oi < i))
                v = jnp.where(better, ov, v)
                i = jnp.where(better, oi, i)
                o = jnp.where(better, oo, o)
            return v, i, o

        pltpu.sync_copy(ids_hbm.at[b], ids_v)
        pltpu.sync_copy(conn_hbm.at[b * _KC + c], conn_v)

        def zinit(k, carry):
            mem_v[pl.ds(k * 16, 16)] = zeros_i
            return carry
        lax.fori_loop(0, nch + 1, zinit, 0)

        def comp(k, st):
            size, offset = st
            v = ids_v[pl.ds(k * 16, 16)]
            meq = v == c
            idxvec = k * 16 + iota16
            ones = jnp.where(meq, 1, 0)
            pre = ones
            for s in (1, 2, 4, 8):
                pre = pre + jnp.where(iota16 >= s,
                                      _shuf(pre, (iota16 - s) & 15), 0)
            pos = size + pre - ones
            plsc.store_scatter(mem_v, [pos], idxvec, mask=meq)
            size = size + _pcnt(meq)
            offset = offset + _pcnt(v < c)
            return size, offset
        size, offset = lax.fori_loop(0, nch, comp,
                                     (jnp.int32(0), jnp.int32(0)))

        def cch(k, st):
            bv, bi = st
            v = conn_v[pl.ds(k * 16, 16)]
            pos = k * 16 + iota16
            m = v > bv
            return jnp.where(m, v, bv), jnp.where(m, pos, bi)
        bv, bi = lax.fori_loop(0, nch, cch, (neg16, zeros_i))
        _, sti = _argmax2(bv, bi)
        start = sti[0]

        def ainit(k, carry):
            cpos = k * 16 + iota16
            mvals = mem_v[pl.ds(k * 16, 16)]
            a = jnp.where((cpos < size) & (mvals != start), 0.0, _NEG)
            alive_v[pl.ds(k * 16, 16)] = a.astype(jnp.float32)
            return carry
        lax.fori_loop(0, nch, ainit, 0)
        plsc.store_scatter(ordseg_v, [zeros_i], zeros_i + start, mask=lane0)

        nck = (size + 15) // 16
        sz16 = nck * 16

        @pl.when((size > 1) & (size <= _CAP))
        def _():
            m0 = mem_v[pl.ds(0, 16)]
            pltpu.async_copy(sim4_hbm.at[b * N + m0[0]], row_v, semA)

            def srow(i2, carry):
                mvec = mem_v[pl.ds(i2 * 16, 16)]
                mnext = mem_v[pl.ds((i2 + 1) * 16, 16)]
                for l in range(16):
                    i = i2 * 16 + l
                    nri = mvec[l + 1] if l < 15 else mnext[0]
                    rb, sb = (row_v, semA) if l % 2 == 0 else (rowb_v, semB)
                    nrb, nsb = (rowb_v, semB) if l % 2 == 0 else (row_v, semA)

                    @pl.when(i + 1 < size)
                    def _issue(nri=nri, nrb=nrb, nsb=nsb):
                        pltpu.async_copy(sim4_hbm.at[b * N + nri], nrb, nsb)

                    @pl.when(i < size)
                    def _compact(i=i, rb=rb, sb=sb):
                        pltpu.make_async_copy(
                            sim4_hbm.at[0], rb, sb).wait()

                        def ccomp(j, carry2):
                            idxs = mem_v[pl.ds(j * 16, 16)]
                            vals = plsc.load_gather(rb, [idxs])
                            subm_v[pl.ds(i * sz16 + j * 16, 16)] = vals
                            return carry2
                        lax.fori_loop(0, nck, ccomp, 0)
                return carry
            lax.fori_loop(0, nck, srow, 0)

            def pscan(j, acc):
                mvals = mem_v[pl.ds(j * 16, 16)]
                cpos = j * 16 + iota16
                return acc + _pcnt((mvals < start) & (cpos < size))
            pstart = lax.fori_loop(0, nck, pscan, jnp.int32(0))

            def step_s(i, cur_p):
                base = cur_p * sz16

                def ch(j, st):
                    sbv, sbi = st
                    vals = (subm_v[pl.ds(base + j * 16, 16)] +
                            alive_v[pl.ds(j * 16, 16)])
                    cpos = j * 16 + iota16
                    m2 = vals > sbv
                    return jnp.where(m2, vals, sbv), jnp.where(m2, cpos, sbi)
                sbv, sbi = lax.fori_loop(0, nck, ch, (neg16, zeros_i))
                _, pv = _argmax2(sbv, sbi)
                p = pv[0]
                mvec2 = mem_v[pl.ds((p // 16) * 16, 16)]
                nxt = _shuf(mvec2, zeros_i + (p % 16))[0]
                plsc.store_scatter(alive_v, [zeros_i + p], neg16, mask=lane0)
                plsc.store_scatter(ordseg_v, [zeros_i + i], zeros_i + nxt,
                                   mask=lane0)
                return p
            lax.fori_loop(1, size, step_s, pstart)

        @pl.when(size > _CAP)
        def _():
            def step(i, cur):
                pltpu.sync_copy(sim4_hbm.at[b * N + cur], row_v)

                def ch(j, st):
                    sbv, sbi, sbo = st
                    idxs = mem_v[pl.ds(j * 16, 16)]
                    av = alive_v[pl.ds(j * 16, 16)]
                    vals = plsc.load_gather(row_v, [idxs]) + av
                    cpos = j * 16 + iota16
                    m = vals > sbv
                    return (jnp.where(m, vals, sbv), jnp.where(m, cpos, sbi),
                            jnp.where(m, idxs, sbo))
                sbv, sbi, sbo = lax.fori_loop(0, nck, ch,
                                              (neg16, zeros_i, zeros_i))
                _, pv, nv = _argmax3(sbv, sbi, sbo)
                p = pv[0]
                nxt = nv[0]
                plsc.store_scatter(alive_v, [zeros_i + p], neg16, mask=lane0)
                plsc.store_scatter(ordseg_v, [zeros_i + i], zeros_i + nxt,
                                   mask=lane0)
                return nxt
            lax.fori_loop(1, size, step, start)

        w = jnp.where(iota16 == 0, offset, jnp.where(iota16 == 1, size, 0))
        offs_v[...] = w.astype(jnp.int32)
        pltpu.sync_copy(offs_v, osz_sh.at[pl.ds(sub * 16, 16)])
        pltpu.sync_copy(ordseg_v, seg_sh.at[pl.ds(sub * N, N)])
        plsc.subcore_barrier()

        @pl.when(c == 0)
        def _():
            pltpu.sync_copy(osz_sh, osz_v)
            pltpu.sync_copy(seg_sh.at[pl.ds(sub * N, _KC * N)], segs_v)
            offs = []
            for cc in range(_KC):
                vv = osz_v[pl.ds((sub + cc) * 16, 16)]
                offs.append(vv[0])
            ovec = zeros_i
            for cc in range(_KC):
                ovec = jnp.where(iota16 == cc, offs[cc], ovec)
            offs_v[...] = ovec

            def asm(k, carry):
                jvec = k * 16 + iota16
                cnt = zeros_i
                for cc in range(_KC):
                    cnt = cnt + jnp.where(jvec >= offs[cc], 1, 0)
                cj = cnt - 1
                offj = plsc.load_gather(offs_v, [cj])
                src = cj * N + jvec - offj
                val = plsc.load_gather(segs_v, [src])
                order_v[pl.ds(k * 16, 16)] = val
                bgidx_v[pl.ds(k * 16, 16)] = val + b * N
                return carry
            lax.fori_loop(0, nch, asm, 0)

            for r in range(N // 128):
                pltpu.async_copy(
                    feat4_hbm.at[bgidx_v.at[pl.ds(r * 128, 128)]],
                    featbuf_v, sem).wait()
                pltpu.sync_copy(featbuf_v,
                                reord_hbm.at[b, pl.ds(r * 128, 128)])
            pltpu.sync_copy(order_v, order_hbm.at[b])

    return sc_kernel(feat4, sim4, conn32, ids)


def kernel(features, coords):
    B, N, D = features.shape
    sim, ids3, conn = _tc_stage(features, coords)
    feat4p = jnp.pad(features.reshape(B * N, D), ((0, 0), (0, 128 - D)))
    reord, order = _sc_greedy(
        feat4p, sim.reshape(B * N, N),
        conn.reshape(B * _KC, N), ids3.reshape(B, N), B, N, D)
    return reord[:, :, :D], order


# speedup vs baseline: 659.9992x; 1.1663x over previous
import functools
import jax
import jax.numpy as jnp
from jax import lax
from jax.experimental import pallas as pl
from jax.experimental.pallas import tpu as pltpu
from jax.experimental.pallas import tpu_sc as plsc

_LAM = 0.5
_SIG2 = 100.0 ** 2
_KC = 8
_NEG = float("-inf")
_CAP = 256
_NBUF = 8


def _tc_body(init_ref, feat_ref, coord_ref, sim_ref, ids_ref, conn_ref,
             cent_ref, conn2_ref, fn_ref):
    feat = feat_ref[0]
    n = feat.shape[0]
    d = feat.shape[1]

    nrm2 = jnp.sum(feat * feat, axis=1, keepdims=True)
    nrm = jnp.maximum(jnp.sqrt(nrm2), 1e-12)
    fnorm = feat / nrm
    fn_ref[...] = fnorm
    c0 = coord_ref[0, :, 0]
    c1 = coord_ref[0, :, 1]

    def simr(r, carry):
        base = pl.multiple_of(r * 128, 128)
        fr = fn_ref[pl.ds(base, 128), :]
        sem = lax.dot_general(fr, fnorm, (((1,), (1,)), ((), ())),
                              preferred_element_type=jnp.float32)
        cr = coord_ref[0, pl.ds(base, 128), :]
        c0r = cr[:, 0]
        c1r = cr[:, 1]
        dx = c0r[:, None] - c0[None, :]
        dy = c1r[:, None] - c1[None, :]
        t = dx * dx + dy * dy
        s = jnp.sqrt(t)
        sp = jnp.exp(-(s * s) / _SIG2)
        sim_ref[0, pl.ds(base, 128), :] = sem + _LAM * sp
        return carry
    lax.fori_loop(0, n // 128, simr, 0)

    for k in range(_KC):
        cent_ref[pl.ds(k, 1), :] = feat_ref[0, pl.ds(init_ref[0, 0, k], 1), :]

    def kiter(it, ids_c):
        best = jnp.full((n,), jnp.inf, jnp.float32)
        bid = jnp.zeros((n,), jnp.int32)
        for c in range(_KC):
            diff = feat - cent_ref[pl.ds(c, 1), :]
            d2 = jnp.sum(diff * diff, axis=1)
            dd = jnp.sqrt(d2)
            upd = dd < best
            best = jnp.where(upd, dd, best)
            bid = jnp.where(upd, c, bid)
        for c in range(_KC):
            m = bid == c
            cnt = jnp.sum(m.astype(jnp.int32))
            ssum = jnp.sum(jnp.where(m[:, None], feat, 0.0), axis=0)
            mean = ssum / jnp.maximum(cnt, 1).astype(jnp.float32)
            old = cent_ref[pl.ds(c, 1), :]
            cent_ref[pl.ds(c, 1), :] = jnp.where(cnt > 0, mean[None, :], old)
        return bid
    ids = lax.fori_loop(0, 10, kiter, jnp.zeros((n,), jnp.int32))
    ids_ref[0, 0, :] = ids

    onehot = (ids[:, None] ==
              lax.broadcasted_iota(jnp.int32, (1, _KC), 1)).astype(jnp.float32)

    def connr(r, carry):
        base = pl.multiple_of(r * 128, 128)
        chunk = sim_ref[0, pl.ds(base, 128), :]
        slab = lax.dot_general(chunk, onehot, (((1,), (0,)), ((), ())),
                               preferred_element_type=jnp.float32)
        conn2_ref[pl.ds(base, 128), :] = slab
        return carry
    lax.fori_loop(0, n // 128, connr, 0)

    connt = jnp.transpose(conn2_ref[...])
    for c in range(_KC):
        m = ids == c
        conn_ref[0, c, :] = jnp.where(m, connt[c], _NEG)


def _init_indices(batch, n):
    rows = []
    for b in range(batch):
        key = jax.random.fold_in(jax.random.key(123), b)
        rows.append(jax.random.permutation(key, n)[:_KC])
    return jnp.stack(rows).astype(jnp.int32).reshape(batch, 1, _KC)


def _tc_stage(features, coords):
    B, N, D = features.shape
    init_idx = _init_indices(B, N)
    sim, ids, conn = pl.pallas_call(
        _tc_body,
        grid=(B,),
        in_specs=[
            pl.BlockSpec((1, 1, _KC), lambda b: (b, 0, 0),
                         memory_space=pltpu.SMEM),
            pl.BlockSpec((1, N, D), lambda b: (b, 0, 0)),
            pl.BlockSpec((1, N, 2), lambda b: (b, 0, 0)),
        ],
        out_specs=[
            pl.BlockSpec((1, N, N), lambda b: (b, 0, 0)),
            pl.BlockSpec((1, 1, N), lambda b: (b, 0, 0)),
            pl.BlockSpec((1, _KC, N), lambda b: (b, 0, 0)),
        ],
        out_shape=[
            jax.ShapeDtypeStruct((B, N, N), jnp.float32),
            jax.ShapeDtypeStruct((B, 1, N), jnp.int32),
            jax.ShapeDtypeStruct((B, _KC, N), jnp.float32),
        ],
        scratch_shapes=[
            pltpu.VMEM((_KC, D), jnp.float32),
            pltpu.VMEM((N, _KC), jnp.float32),
            pltpu.VMEM((N, D), jnp.float32),
        ],
    )(init_idx, features, coords)
    return sim, ids, conn


def _sc_greedy(feat4, sim4, conn32, ids, B, N, D):
    mesh = plsc.VectorSubcoreMesh(core_axis_name="c", subcore_axis_name="s",
                                  num_cores=2, num_subcores=16)
    nch = N // 16

    @functools.partial(
        pl.kernel,
        mesh=mesh,
        compiler_params=pltpu.CompilerParams(needs_layout_passes=False),
        out_type=[
            jax.ShapeDtypeStruct((B, N, 128), jnp.float32),
            jax.ShapeDtypeStruct((B, N), jnp.int32),
        ],
        scratch_types=[
            pltpu.VMEM((N,), jnp.int32),
            pltpu.VMEM((N + 16,), jnp.int32),
            pltpu.VMEM((N,), jnp.float32),
            pltpu.VMEM((_NBUF * N,), jnp.float32),
            pltpu.VMEM((N,), jnp.float32),
            pltpu.VMEM((N,), jnp.int32),
            pltpu.VMEM((16,), jnp.int32),
            pltpu.VMEM((_KC * N,), jnp.int32),
            pltpu.VMEM((16 * 16,), jnp.int32),
            pltpu.VMEM((N,), jnp.int32),
            pltpu.VMEM((N,), jnp.int32),
            pltpu.VMEM((128, 128), jnp.float32),
            pltpu.VMEM((128, 128), jnp.float32),
            pltpu.VMEM((_CAP * _CAP,), jnp.float32),
            pltpu.VMEM_SHARED((16 * N,), jnp.int32),
            pltpu.VMEM_SHARED((16 * 16,), jnp.int32),
            pltpu.SemaphoreType.DMA,
            pltpu.SemaphoreType.DMA,
        ] + [pltpu.SemaphoreType.DMA] * _NBUF,
    )
    def sc_kernel(feat4_hbm, sim4_hbm, conn_hbm, ids_hbm, reord_hbm,
                  order_hbm, ids_v, mem_v, conn_v, rows8_v, alive_v,
                  ordseg_v, offs_v, segs_v, osz_v, order_v, bgidx_v,
                  featbuf_v, featbuf2_v, subm_v, seg_sh, osz_sh,
                  semA, semB, *sems):
        core = lax.axis_index("c")
        sub = lax.axis_index("s")
        b = core * 2 + sub // _KC
        c = sub % _KC
        iota16 = lax.broadcasted_iota(jnp.int32, (16,), 0)
        zeros_i = jnp.zeros((16,), jnp.int32)
        neg16 = jnp.full((16,), _NEG, jnp.float32)
        lane0 = iota16 == 0
        shufs = [iota16 ^ 8, iota16 ^ 4, iota16 ^ 2, iota16 ^ 1]

        def _shuf(x, idx):
            return x.at[idx].get(mode="promise_in_bounds")

        def _pcnt(m):
            r = plsc.all_reduce_population_count(m)
            return r if getattr(r, "ndim", 0) == 0 else r[0]

        def _argmax2(v, i):
            for s in shufs:
                ov, oi = _shuf(v, s), _shuf(i, s)
                better = (ov > v) | ((ov == v) & (oi < i))
                v = jnp.where(better, ov, v)
                i = jnp.where(better, oi, i)
            return v, i

        def _argmax3(v, i, o):
            for s in shufs:
                ov, oi, oo = _shuf(v, s), _shuf(i, s), _shuf(o, s)
                better = (ov > v) | ((ov == v) & (oi < i))
                v = jnp.where(better, ov, v)
                i = jnp.where(better, oi, i)
                o = jnp.where(better, oo, o)
            return v, i, o

        pltpu.sync_copy(ids_hbm.at[b], ids_v)
        pltpu.sync_copy(conn_hbm.at[b * _KC + c], conn_v)

        def zinit(k, carry):
            mem_v[pl.ds(k * 16, 16)] = zeros_i
            return carry
        lax.fori_loop(0, nch + 1, zinit, 0)

        def comp(k, st):
            size, offset = st
            v = ids_v[pl.ds(k * 16, 16)]
            meq = v == c
            idxvec = k * 16 + iota16
            ones = jnp.where(meq, 1, 0)
            pre = ones
            for s in (1, 2, 4, 8):
                pre = pre + jnp.where(iota16 >= s,
                                      _shuf(pre, (iota16 - s) & 15), 0)
            pos = size + pre - ones
            plsc.store_scatter(mem_v, [pos], idxvec, mask=meq)
            size = size + _pcnt(meq)
            offset = offset + _pcnt(v < c)
            return size, offset
        size, offset = lax.fori_loop(0, nch, comp,
                                     (jnp.int32(0), jnp.int32(0)))

        def cch(k, st):
            bv, bi = st
            v = conn_v[pl.ds(k * 16, 16)]
            pos = k * 16 + iota16
            m = v > bv
            return jnp.where(m, v, bv), jnp.where(m, pos, bi)
        bv, bi = lax.fori_loop(0, nch, cch, (neg16, zeros_i))
        _, sti = _argmax2(bv, bi)
        start = sti[0]

        def ainit(k, carry):
            cpos = k * 16 + iota16
            mvals = mem_v[pl.ds(k * 16, 16)]
            a = jnp.where((cpos < size) & (mvals != start), 0.0, _NEG)
            alive_v[pl.ds(k * 16, 16)] = a.astype(jnp.float32)
            return carry
        lax.fori_loop(0, nch, ainit, 0)
        plsc.store_scatter(ordseg_v, [zeros_i], zeros_i + start, mask=lane0)

        nck = (size + 15) // 16
        sz16 = nck * 16

        @pl.when((size > 1) & (size <= _CAP))
        def _():
            m0 = mem_v[pl.ds(0, 16)]
            for q in range(_NBUF):
                @pl.when(q < size)
                def _prime(q=q):
                    pltpu.async_copy(
                        sim4_hbm.at[b * N + m0[q]],
                        rows8_v.at[pl.ds(q * N, N)], sems[q])

            def srow(i2, carry):
                mvec = mem_v[pl.ds(i2 * 16, 16)]
                mnext = mem_v[pl.ds((i2 + 1) * 16, 16)]
                for l in range(16):
                    i = i2 * 16 + l
                    s = l % _NBUF
                    nri = mvec[l + _NBUF] if l < _NBUF else mnext[l - _NBUF]

                    @pl.when(i < size)
                    def _compact(i=i, s=s):
                        pltpu.make_async_copy(
                            sim4_hbm.at[0],
                            rows8_v.at[pl.ds(s * N, N)], sems[s]).wait()
                        off = jnp.int32(s * N)

                        def ccomp(j, carry2):
                            idxs = mem_v[pl.ds(j * 16, 16)]
                            vals = plsc.load_gather(rows8_v, [idxs + off])
                            subm_v[pl.ds(i * sz16 + j * 16, 16)] = vals
                            return carry2
                        lax.fori_loop(0, nck, ccomp, 0)

                    @pl.when(i + _NBUF < size)
                    def _issue(nri=nri, s=s):
                        pltpu.async_copy(
                            sim4_hbm.at[b * N + nri],
                            rows8_v.at[pl.ds(s * N, N)], sems[s])
                return carry
            lax.fori_loop(0, nck, srow, 0)

            def pscan(j, acc):
                mvals = mem_v[pl.ds(j * 16, 16)]
                cpos = j * 16 + iota16
                return acc + _pcnt((mvals < start) & (cpos < size))
            pstart = lax.fori_loop(0, nck, pscan, jnp.int32(0))

            def step_s(i, cur_p):
                base = cur_p * sz16

                def ch(j, st):
                    sbv, sbi = st
                    vals = (subm_v[pl.ds(base + j * 16, 16)] +
                            alive_v[pl.ds(j * 16, 16)])
                    cpos = j * 16 + iota16
                    m2 = vals > sbv
                    return jnp.where(m2, vals, sbv), jnp.where(m2, cpos, sbi)
                sbv, sbi = lax.fori_loop(0, nck, ch, (neg16, zeros_i))
                _, pv = _argmax2(sbv, sbi)
                p = pv[0]
                mvec2 = mem_v[pl.ds((p // 16) * 16, 16)]
                nxt = _shuf(mvec2, zeros_i + (p % 16))[0]
                plsc.store_scatter(alive_v, [zeros_i + p], neg16, mask=lane0)
                plsc.store_scatter(ordseg_v, [zeros_i + i], zeros_i + nxt,
                                   mask=lane0)
                return p
            lax.fori_loop(1, size, step_s, pstart)

        @pl.when(size > _CAP)
        def _():
            def step(i, cur):
                pltpu.sync_copy(sim4_hbm.at[b * N + cur],
                                rows8_v.at[pl.ds(0, N)])

                def ch(j, st):
                    sbv, sbi, sbo = st
                    idxs = mem_v[pl.ds(j * 16, 16)]
                    av = alive_v[pl.ds(j * 16, 16)]
                    vals = plsc.load_gather(rows8_v, [idxs]) + av
                    cpos = j * 16 + iota16
                    m = vals > sbv
                    return (jnp.where(m, vals, sbv), jnp.where(m, cpos, sbi),
                            jnp.where(m, idxs, sbo))
                sbv, sbi, sbo = lax.fori_loop(0, nck, ch,
                                              (neg16, zeros_i, zeros_i))
                _, pv, nv = _argmax3(sbv, sbi, sbo)
                p = pv[0]
                nxt = nv[0]
                plsc.store_scatter(alive_v, [zeros_i + p], neg16, mask=lane0)
                plsc.store_scatter(ordseg_v, [zeros_i + i], zeros_i + nxt,
                                   mask=lane0)
                return nxt
            lax.fori_loop(1, size, step, start)

        w = jnp.where(iota16 == 0, offset, jnp.where(iota16 == 1, size, 0))
        offs_v[...] = w.astype(jnp.int32)
        pltpu.sync_copy(offs_v, osz_sh.at[pl.ds(sub * 16, 16)])
        pltpu.sync_copy(ordseg_v, seg_sh.at[pl.ds(sub * N, N)])
        plsc.subcore_barrier()

        @pl.when(c == 0)
        def _():
            pltpu.sync_copy(osz_sh, osz_v)
            pltpu.sync_copy(seg_sh.at[pl.ds(sub * N, _KC * N)], segs_v)
            offs = []
            for cc in range(_KC):
                vv = osz_v[pl.ds((sub + cc) * 16, 16)]
                offs.append(vv[0])
            ovec = zeros_i
            for cc in range(_KC):
                ovec = jnp.where(iota16 == cc, offs[cc], ovec)
            offs_v[...] = ovec

            def asm(k, carry):
                jvec = k * 16 + iota16
                cnt = zeros_i
                for cc in range(_KC):
                    cnt = cnt + jnp.where(jvec >= offs[cc], 1, 0)
                cj = cnt - 1
                offj = plsc.load_gather(offs_v, [cj])
                src = cj * N + jvec - offj
                val = plsc.load_gather(segs_v, [src])
                order_v[pl.ds(k * 16, 16)] = val
                bgidx_v[pl.ds(k * 16, 16)] = val + b * N
                return carry
            lax.fori_loop(0, nch, asm, 0)

            pltpu.async_copy(
                feat4_hbm.at[bgidx_v.at[pl.ds(0, 128)]], featbuf_v, semA)
            for r in range(N // 128):
                buf, sm = (featbuf_v, semA) if r % 2 == 0 else (featbuf2_v,
                                                                semB)
                if r + 1 < N // 128:
                    nbuf, nsm = ((featbuf2_v, semB) if r % 2 == 0
                                 else (featbuf_v, semA))
                    pltpu.async_copy(
                        feat4_hbm.at[bgidx_v.at[pl.ds((r + 1) * 128, 128)]],
                        nbuf, nsm)
                pltpu.make_async_copy(
                    feat4_hbm.at[pl.ds(0, 128)], buf, sm).wait()
                pltpu.sync_copy(buf, reord_hbm.at[b, pl.ds(r * 128, 128)])
            pltpu.sync_copy(order_v, order_hbm.at[b])

    return sc_kernel(feat4, sim4, conn32, ids)


def kernel(features, coords):
    B, N, D = features.shape
    sim, ids3, conn = _tc_stage(features, coords)
    feat4p = jnp.pad(features.reshape(B * N, D), ((0, 0), (0, 128 - D)))
    reord, order = _sc_greedy(
        feat4p, sim.reshape(B * N, N),
        conn.reshape(B * _KC, N), ids3.reshape(B, N), B, N, D)
    return reord[:, :, :D], order
